# Initial kernel scaffold; baseline (speedup 1.0000x reference)
import functools

import jax
import jax.numpy as jnp
from jax import lax
from jax.experimental import pallas as pl
from jax.experimental.pallas import tpu as pltpu

RING_ID_START = 4
RING_ID_END = 104
NSLOT = 128
B, S, E, H = 16, 2048, 256, 64
ROUT = RING_ID_END - RING_ID_START


def _logits_body(x_ref, xg_ref, m_ref, w0_ref, b0_ref, w1_ref, b1_ref,
                 out_ref):
    scale = H ** -0.5
    xg = xg_ref[0]
    m = m_ref[...]
    out1t = jnp.dot(xg, w1_ref[...], preferred_element_type=jnp.float32,
                    precision=lax.Precision.HIGHEST) + b1_ref[...]
    n = lax.dot_general(w0_ref[...], out1t, (((1,), (1,)), ((), ())),
                        preferred_element_type=jnp.float32,
                        precision=lax.Precision.HIGHEST) * m
    c = lax.dot_general(b0_ref[...], out1t, (((0,), (1,)), ((), ())),
                        preferred_element_type=jnp.float32,
                        precision=lax.Precision.HIGHEST)[None, :] * m
    xt = x_ref[0]
    acc = jnp.dot(xt, n, preferred_element_type=jnp.float32,
                  precision=lax.Precision.HIGHEST)
    out = scale * (acc + c)
    out_ref[0] = out[:, :ROUT]


def _logits_call(x, xg, maskf, W0, b0, W1, b1, interpret=False):
    return pl.pallas_call(
        _logits_body,
        grid=(B,),
        in_specs=[
            pl.BlockSpec((1, S, E), lambda b: (b, 0, 0)),
            pl.BlockSpec((1, NSLOT, E), lambda b: (b, 0, 0)),
            pl.BlockSpec((1, NSLOT), lambda b: (b, 0)),
            pl.BlockSpec((E, H), lambda b: (0, 0)),
            pl.BlockSpec((H,), lambda b: (0,)),
            pl.BlockSpec((E, H), lambda b: (0, 0)),
            pl.BlockSpec((H,), lambda b: (0,)),
        ],
        out_specs=pl.BlockSpec((1, S, ROUT), lambda b: (b, 0, 0)),
        out_shape=jax.ShapeDtypeStruct((B, S, ROUT), jnp.float32),
        interpret=interpret,
    )(x, xg, maskf, W0, b0, W1, b1)


def kernel(x, sequences, W0, b0, W1, b1):
    slot = sequences - RING_ID_START
    valid = (sequences >= RING_ID_START) & (sequences <= RING_ID_END - 1)
    slot = jnp.where(valid, slot, NSLOT - 1)
    s_iota = jnp.broadcast_to(jnp.arange(S, dtype=jnp.int32)[None, :], (B, S))
    winner = jnp.full((B, NSLOT), -1, jnp.int32)
    winner = winner.at[jnp.arange(B)[:, None], slot].max(
        jnp.where(valid, s_iota, -1))
    winner = winner.at[:, ROUT:].set(-1)
    maskf = (winner >= 0).astype(jnp.float32)
    flat_idx = (jnp.maximum(winner, 0)
                + jnp.arange(B, dtype=jnp.int32)[:, None] * S)
    xg = x.reshape(B * S, E)[flat_idx.reshape(-1)].reshape(B, NSLOT, E)
    return _logits_call(x, xg, maskf, W0, b0, W1, b1)


# speedup vs baseline: 1.0983x; 1.0983x over previous
import functools

import jax
import jax.numpy as jnp
from jax import lax
from jax.experimental import pallas as pl
from jax.experimental.pallas import tpu as pltpu

RING_ID_START = 4
RING_ID_END = 104
NSLOT = 128
B, S, E, H = 16, 2048, 256, 64
ROUT = RING_ID_END - RING_ID_START


def _logits_body(x_ref, xg_ref, m_ref, w0_ref, b0_ref, w1_ref, b1_ref,
                 out_ref):
    scale = H ** -0.5
    xg = xg_ref[0]
    m = m_ref[0]
    out1t = jnp.dot(xg, w1_ref[...], preferred_element_type=jnp.float32,
                    precision=lax.Precision.HIGHEST) + b1_ref[...]
    n = lax.dot_general(w0_ref[...], out1t, (((1,), (1,)), ((), ())),
                        preferred_element_type=jnp.float32,
                        precision=lax.Precision.HIGHEST) * m
    c = lax.dot_general(b0_ref[...], out1t, (((0,), (1,)), ((), ())),
                        preferred_element_type=jnp.float32,
                        precision=lax.Precision.HIGHEST)[None, :] * m
    xt = x_ref[0]
    acc = jnp.dot(xt, n, preferred_element_type=jnp.float32,
                  precision=lax.Precision.HIGHEST)
    out = scale * (acc + c)
    out_ref[0] = out[:, :ROUT]


def _logits_call(x, xg, maskf, W0, b0, W1, b1, interpret=False):
    return pl.pallas_call(
        _logits_body,
        grid=(B,),
        in_specs=[
            pl.BlockSpec((1, S, E), lambda b: (b, 0, 0)),
            pl.BlockSpec((1, NSLOT, E), lambda b: (b, 0, 0)),
            pl.BlockSpec((1, 1, NSLOT), lambda b: (b, 0, 0)),
            pl.BlockSpec((E, H), lambda b: (0, 0)),
            pl.BlockSpec((H,), lambda b: (0,)),
            pl.BlockSpec((E, H), lambda b: (0, 0)),
            pl.BlockSpec((H,), lambda b: (0,)),
        ],
        out_specs=pl.BlockSpec((1, S, ROUT), lambda b: (b, 0, 0)),
        out_shape=jax.ShapeDtypeStruct((B, S, ROUT), jnp.float32),
        interpret=interpret,
    )(x, xg, maskf, W0, b0, W1, b1)


def kernel(x, sequences, W0, b0, W1, b1):
    slot = sequences - RING_ID_START
    valid = (sequences >= RING_ID_START) & (sequences <= RING_ID_END - 1)
    slot = jnp.where(valid, slot, NSLOT - 1)
    s_iota = jnp.broadcast_to(jnp.arange(S, dtype=jnp.int32)[None, :], (B, S))
    winner = jnp.full((B, NSLOT), -1, jnp.int32)
    winner = winner.at[jnp.arange(B)[:, None], slot].max(
        jnp.where(valid, s_iota, -1))
    winner = winner.at[:, ROUT:].set(-1)
    maskf = (winner >= 0).astype(jnp.float32).reshape(B, 1, NSLOT)
    flat_idx = (jnp.maximum(winner, 0)
                + jnp.arange(B, dtype=jnp.int32)[:, None] * S)
    xg = x.reshape(B * S, E)[flat_idx.reshape(-1)].reshape(B, NSLOT, E)
    return _logits_call(x, xg, maskf, W0, b0, W1, b1)


# speedup vs baseline: 1.7753x; 1.6164x over previous
import functools

import jax
import jax.numpy as jnp
from jax import lax
from jax.experimental import pallas as pl
from jax.experimental.pallas import tpu as pltpu
from jax.experimental.pallas import tpu_sc as plsc

RING_ID_START = 4
RING_ID_END = 104
NSLOT = 128
B, S, E, H = 16, 2048, 256, 64
ROUT = RING_ID_END - RING_ID_START

_NC, _NS = 2, 16
_NW = _NC * _NS
_BPW = (B * NSLOT) // _NW


def _winner_body(seq_ref, flat_ref, mask_ref):
    for b in range(B):
        row = seq_ref[b, :]
        rowb = jnp.broadcast_to(row[None, :], (NSLOT, S))
        valid = (rowb >= RING_ID_START) & (rowb <= RING_ID_END - 1)
        slotb = rowb - RING_ID_START
        jcol = lax.broadcasted_iota(jnp.int32, (NSLOT, S), 0)
        siota = lax.broadcasted_iota(jnp.int32, (NSLOT, S), 1)
        vals = jnp.where((slotb == jcol) & valid, siota, -1)
        winner = jnp.max(vals, axis=1)
        flat_ref[b, :] = jnp.maximum(winner, 0) + b * S
        mask_ref[b, 0, :] = (winner >= 0).astype(jnp.float32)


def _winner_call(sequences):
    return pl.pallas_call(
        _winner_body,
        out_shape=(
            jax.ShapeDtypeStruct((B, NSLOT), jnp.int32),
            jax.ShapeDtypeStruct((B, 1, NSLOT), jnp.float32),
        ),
    )(sequences)


def _gather_sc_body(table_hbm, idx_hbm, out_hbm, idx_v, rows_v, sem):
    wid = lax.axis_index("s") * _NC + lax.axis_index("c")
    base = wid * _BPW
    pltpu.sync_copy(idx_hbm.at[pl.ds(base, _BPW)], idx_v)
    pltpu.async_copy(table_hbm.at[idx_v], rows_v, sem).wait()
    pltpu.sync_copy(rows_v, out_hbm.at[pl.ds(base, _BPW)])


@functools.cache
def _gather_sc_kernel():
    return pl.kernel(
        _gather_sc_body,
        mesh=plsc.VectorSubcoreMesh(core_axis_name="c", subcore_axis_name="s"),
        out_type=jax.ShapeDtypeStruct((B * NSLOT, E), jnp.float32),
        scratch_types=[
            pltpu.VMEM((_BPW,), jnp.int32),
            pltpu.VMEM((_BPW, E), jnp.float32),
            pltpu.SemaphoreType.DMA,
        ],
    )


def _gather_sc(table, idx):
    return _gather_sc_kernel()(table, idx)


def _logits_body(x_ref, xg_ref, m_ref, w0_ref, b0_ref, w1_ref, b1_ref,
                 out_ref):
    scale = H ** -0.5
    xg = xg_ref[0]
    m = m_ref[0]
    out1t = jnp.dot(xg, w1_ref[...], preferred_element_type=jnp.float32,
                    precision=lax.Precision.HIGHEST) + b1_ref[...]
    n = lax.dot_general(w0_ref[...], out1t, (((1,), (1,)), ((), ())),
                        preferred_element_type=jnp.float32,
                        precision=lax.Precision.HIGHEST) * m
    c = lax.dot_general(b0_ref[...], out1t, (((0,), (1,)), ((), ())),
                        preferred_element_type=jnp.float32,
                        precision=lax.Precision.HIGHEST)[None, :] * m
    xt = x_ref[0]
    acc = jnp.dot(xt, n, preferred_element_type=jnp.float32,
                  precision=lax.Precision.HIGHEST)
    out = scale * (acc + c)
    out_ref[0] = out[:, :ROUT]


def _logits_call(x, xg, maskf, W0, b0, W1, b1):
    return pl.pallas_call(
        _logits_body,
        grid=(B,),
        in_specs=[
            pl.BlockSpec((1, S, E), lambda b: (b, 0, 0)),
            pl.BlockSpec((1, NSLOT, E), lambda b: (b, 0, 0)),
            pl.BlockSpec((1, 1, NSLOT), lambda b: (b, 0, 0)),
            pl.BlockSpec((E, H), lambda b: (0, 0)),
            pl.BlockSpec((H,), lambda b: (0,)),
            pl.BlockSpec((E, H), lambda b: (0, 0)),
            pl.BlockSpec((H,), lambda b: (0,)),
        ],
        out_specs=pl.BlockSpec((1, S, ROUT), lambda b: (b, 0, 0)),
        out_shape=jax.ShapeDtypeStruct((B, S, ROUT), jnp.float32),
    )(x, xg, maskf, W0, b0, W1, b1)


def kernel(x, sequences, W0, b0, W1, b1):
    flat_idx, maskf = _winner_call(sequences)
    xg = _gather_sc(x.reshape(B * S, E), flat_idx.reshape(B * NSLOT))
    return _logits_call(x, xg.reshape(B, NSLOT, E), maskf, W0, b0, W1, b1)


# speedup vs baseline: 1.9128x; 1.0774x over previous
import functools

import jax
import jax.numpy as jnp
from jax import lax
from jax.experimental import pallas as pl
from jax.experimental.pallas import tpu as pltpu
from jax.experimental.pallas import tpu_sc as plsc

RING_ID_START = 4
RING_ID_END = 104
NSLOT = 128
B, S, E, H = 16, 2048, 256, 64
ROUT = RING_ID_END - RING_ID_START

_NC, _NS = 2, 16
_NW = _NC * _NS
_BPW = (B * NSLOT) // _NW


def _winner_body(seq_ref, flat_ref, mask_ref):
    for b in range(B):
        row = seq_ref[b, :]
        rowb = jnp.broadcast_to(row[None, :], (NSLOT, S))
        valid = (rowb >= RING_ID_START) & (rowb <= RING_ID_END - 1)
        slotb = rowb - RING_ID_START
        jcol = lax.broadcasted_iota(jnp.int32, (NSLOT, S), 0)
        siota = lax.broadcasted_iota(jnp.int32, (NSLOT, S), 1)
        vals = jnp.where((slotb == jcol) & valid, siota, -1)
        winner = jnp.max(vals, axis=1)
        flat_ref[b, :] = jnp.maximum(winner, 0) + b * S
        mask_ref[b, 0, :] = (winner >= 0).astype(jnp.float32)


def _winner_call(sequences):
    return pl.pallas_call(
        _winner_body,
        out_shape=(
            jax.ShapeDtypeStruct((B, NSLOT), jnp.int32),
            jax.ShapeDtypeStruct((B, 1, NSLOT), jnp.float32),
        ),
    )(sequences)


def _gather_sc_body(table_hbm, idx_hbm, out_hbm, idx_v, rows_v, sem):
    wid = lax.axis_index("s") * _NC + lax.axis_index("c")
    base = wid * _BPW
    pltpu.sync_copy(idx_hbm.at[pl.ds(base, _BPW)], idx_v)
    pltpu.async_copy(table_hbm.at[idx_v], rows_v, sem).wait()
    pltpu.sync_copy(rows_v, out_hbm.at[pl.ds(base, _BPW)])


@functools.cache
def _gather_sc_kernel():
    return pl.kernel(
        _gather_sc_body,
        mesh=plsc.VectorSubcoreMesh(core_axis_name="c", subcore_axis_name="s"),
        out_type=jax.ShapeDtypeStruct((B * NSLOT, E), jnp.float32),
        scratch_types=[
            pltpu.VMEM((_BPW,), jnp.int32),
            pltpu.VMEM((_BPW, E), jnp.float32),
            pltpu.SemaphoreType.DMA,
        ],
    )


def _gather_sc(table, idx):
    return _gather_sc_kernel()(table, idx)


def _logits_body(x_ref, xg_ref, m_ref, w0_ref, b0_ref, w1_ref, b1_ref,
                 out_ref):
    scale = H ** -0.5
    xg = xg_ref[0]
    m_t = m_ref[0]
    out1t = (jnp.dot(xg, w1_ref[...], preferred_element_type=jnp.float32,
                     precision=lax.Precision.HIGHEST) + b1_ref[...]) * m_t
    n = lax.dot_general(w0_ref[...], out1t, (((1,), (1,)), ((), ())),
                        preferred_element_type=jnp.float32,
                        precision=lax.Precision.HIGHEST)
    c_t = jnp.dot(out1t, b0_ref[...], preferred_element_type=jnp.float32,
                  precision=lax.Precision.HIGHEST)
    acc_t = lax.dot_general(n, x_ref[0], (((0,), (1,)), ((), ())),
                            preferred_element_type=jnp.float32)
    res = scale * (acc_t + c_t)
    out_ref[...] = res[:ROUT, :]


def _logits_call(x, xg, maskf, W0, b0, W1, b1):
    out = pl.pallas_call(
        _logits_body,
        grid=(B,),
        in_specs=[
            pl.BlockSpec((1, S, E), lambda b: (b, 0, 0)),
            pl.BlockSpec((1, NSLOT, E), lambda b: (b, 0, 0)),
            pl.BlockSpec((1, NSLOT, 1), lambda b: (b, 0, 0)),
            pl.BlockSpec((E, H), lambda b: (0, 0)),
            pl.BlockSpec((H, 1), lambda b: (0, 0)),
            pl.BlockSpec((E, H), lambda b: (0, 0)),
            pl.BlockSpec((H,), lambda b: (0,)),
        ],
        out_specs=pl.BlockSpec((ROUT, S), lambda b: (0, b)),
        out_shape=jax.ShapeDtypeStruct((ROUT, B * S), jnp.float32),
    )(x, xg, maskf, W0, b0.reshape(H, 1), W1, b1)
    return jnp.transpose(out.reshape(ROUT, B, S), (1, 2, 0))


def kernel(x, sequences, W0, b0, W1, b1):
    flat_idx, maskf = _winner_call(sequences)
    xg = _gather_sc(x.reshape(B * S, E), flat_idx.reshape(B * NSLOT))
    return _logits_call(x, xg.reshape(B, NSLOT, E),
                        maskf.reshape(B, NSLOT, 1), W0, b0, W1, b1)


# speedup vs baseline: 2.1250x; 1.1110x over previous
import functools

import jax
import jax.numpy as jnp
from jax import lax
from jax.experimental import pallas as pl
from jax.experimental.pallas import tpu as pltpu
from jax.experimental.pallas import tpu_sc as plsc

RING_ID_START = 4
RING_ID_END = 104
NSLOT = 128
B, S, E, H = 16, 2048, 256, 64
ROUT = RING_ID_END - RING_ID_START

_NC, _NS = 2, 16
_NW = _NC * _NS
_BPW = (B * NSLOT) // _NW


def _winner_body(seq_ref, flat_ref, mask_ref):
    for b in range(B):
        row = seq_ref[b, :]
        valid = (row >= RING_ID_START) & (row <= RING_ID_END - 1)
        slot = jnp.where(valid, row - RING_ID_START, -1)
        slotb = jnp.broadcast_to(slot[None, :], (NSLOT, S))
        jcol = lax.broadcasted_iota(jnp.int32, (NSLOT, S), 0)
        siota = lax.broadcasted_iota(jnp.int32, (NSLOT, S), 1)
        vals = jnp.where(slotb == jcol, siota, -1)
        winner = jnp.max(vals, axis=1)
        flat_ref[b, :] = jnp.maximum(winner, 0) + b * S
        mask_ref[b, 0, :] = (winner >= 0).astype(jnp.float32)


def _winner_call(sequences):
    return pl.pallas_call(
        _winner_body,
        out_shape=(
            jax.ShapeDtypeStruct((B, NSLOT), jnp.int32),
            jax.ShapeDtypeStruct((B, 1, NSLOT), jnp.float32),
        ),
    )(sequences)


def _gather_sc_body(table_hbm, idx_hbm, out_hbm, idx_v, rows_v, sem):
    wid = lax.axis_index("s") * _NC + lax.axis_index("c")
    base = wid * _BPW
    pltpu.sync_copy(idx_hbm.at[pl.ds(base, _BPW)], idx_v)
    pltpu.async_copy(table_hbm.at[idx_v], rows_v, sem).wait()
    pltpu.sync_copy(rows_v, out_hbm.at[pl.ds(base, _BPW)])


@functools.cache
def _gather_sc_kernel():
    return pl.kernel(
        _gather_sc_body,
        mesh=plsc.VectorSubcoreMesh(core_axis_name="c", subcore_axis_name="s"),
        out_type=jax.ShapeDtypeStruct((B * NSLOT, E), jnp.float32),
        scratch_types=[
            pltpu.VMEM((_BPW,), jnp.int32),
            pltpu.VMEM((_BPW, E), jnp.float32),
            pltpu.SemaphoreType.DMA,
        ],
    )


def _gather_sc(table, idx):
    return _gather_sc_kernel()(table, idx)


def _logits_body(x_ref, xg_ref, m_ref, w0t_ref, b0_ref, w1t_ref, b1_ref,
                 out_ref):
    scale = H ** -0.5
    xg = xg_ref[0]
    m = m_ref[0]
    out1t = lax.dot_general(xg, w1t_ref[...], (((1,), (1,)), ((), ())),
                            preferred_element_type=jnp.float32,
                            precision=lax.Precision.HIGHEST) + b1_ref[...]
    n = lax.dot_general(w0t_ref[...], out1t, (((0,), (1,)), ((), ())),
                        preferred_element_type=jnp.float32,
                        precision=lax.Precision.HIGHEST) * m
    c = lax.dot_general(b0_ref[...], out1t, (((0,), (1,)), ((), ())),
                        preferred_element_type=jnp.float32,
                        precision=lax.Precision.HIGHEST)[None, :] * m
    acc = jnp.dot(x_ref[0], n, preferred_element_type=jnp.float32)
    out = scale * (acc + c)
    out_ref[0] = out[:, :ROUT]


def _logits_call(x, xg, maskf, W0, b0, W1, b1):
    return pl.pallas_call(
        _logits_body,
        grid=(B,),
        in_specs=[
            pl.BlockSpec((1, S, E), lambda b: (b, 0, 0)),
            pl.BlockSpec((1, NSLOT, E), lambda b: (b, 0, 0)),
            pl.BlockSpec((1, 1, NSLOT), lambda b: (b, 0, 0)),
            pl.BlockSpec((H, E), lambda b: (0, 0)),
            pl.BlockSpec((H,), lambda b: (0,)),
            pl.BlockSpec((H, E), lambda b: (0, 0)),
            pl.BlockSpec((H,), lambda b: (0,)),
        ],
        out_specs=pl.BlockSpec((1, S, ROUT), lambda b: (b, 0, 0)),
        out_shape=jax.ShapeDtypeStruct((B, S, ROUT), jnp.float32),
    )(x, xg, maskf, jnp.transpose(W0), b0, jnp.transpose(W1), b1)


def kernel(x, sequences, W0, b0, W1, b1):
    flat_idx, maskf = _winner_call(sequences)
    xg = _gather_sc(x.reshape(B * S, E), flat_idx.reshape(B * NSLOT))
    return _logits_call(x, xg.reshape(B, NSLOT, E), maskf, W0, b0, W1, b1)


# speedup vs baseline: 2.2245x; 1.0468x over previous
import functools

import jax
import jax.numpy as jnp
from jax import lax
from jax.experimental import pallas as pl
from jax.experimental.pallas import tpu as pltpu
from jax.experimental.pallas import tpu_sc as plsc

RING_ID_START = 4
RING_ID_END = 104
NSLOT = 128
B, S, E, H = 16, 2048, 256, 64
ROUT = RING_ID_END - RING_ID_START

_NC, _NS = 2, 16
_NW = _NC * _NS
_BPW = (B * NSLOT) // _NW


def _winner_body(seq_ref, flat_ref, mask_ref):
    for b in range(B):
        row = seq_ref[b, :]
        valid = (row >= RING_ID_START) & (row <= RING_ID_END - 1)
        slot = jnp.where(valid, row - RING_ID_START, -1)
        slotb = jnp.broadcast_to(slot[None, :], (NSLOT, S))
        jcol = lax.broadcasted_iota(jnp.int32, (NSLOT, S), 0)
        siota = lax.broadcasted_iota(jnp.int32, (NSLOT, S), 1)
        vals = jnp.where(slotb == jcol, siota, -1)
        winner = jnp.max(vals, axis=1)
        flat_ref[b, :] = jnp.maximum(winner, 0) + b * S
        mask_ref[b, 0, :] = (winner >= 0).astype(jnp.float32)


def _winner_call(sequences):
    return pl.pallas_call(
        _winner_body,
        out_shape=(
            jax.ShapeDtypeStruct((B, NSLOT), jnp.int32),
            jax.ShapeDtypeStruct((B, 1, NSLOT), jnp.float32),
        ),
    )(sequences)


def _gather_sc_body(table_hbm, idx_hbm, out_hbm, idx_v, rows_v, sem):
    wid = lax.axis_index("s") * _NC + lax.axis_index("c")
    base = wid * _BPW
    pltpu.sync_copy(idx_hbm.at[pl.ds(base, _BPW)], idx_v)
    pltpu.async_copy(table_hbm.at[idx_v], rows_v, sem).wait()
    pltpu.sync_copy(rows_v, out_hbm.at[pl.ds(base, _BPW)])


@functools.cache
def _gather_sc_kernel():
    return pl.kernel(
        _gather_sc_body,
        mesh=plsc.VectorSubcoreMesh(core_axis_name="c", subcore_axis_name="s"),
        out_type=jax.ShapeDtypeStruct((B * NSLOT, E), jnp.float32),
        scratch_types=[
            pltpu.VMEM((_BPW,), jnp.int32),
            pltpu.VMEM((_BPW, E), jnp.float32),
            pltpu.SemaphoreType.DMA,
        ],
    )


def _gather_sc(table, idx):
    return _gather_sc_kernel()(table, idx)


def _logits_body(x_ref, xg_ref, m_ref, w0t_ref, b0_ref, w1t_ref, b1_ref,
                 out_ref):
    scale = H ** -0.5
    xg = xg_ref[0]
    m = m_ref[0]
    out1t = lax.dot_general(xg, w1t_ref[...], (((1,), (1,)), ((), ())),
                            preferred_element_type=jnp.float32,
                            precision=lax.Precision.HIGHEST) + b1_ref[...]
    n = lax.dot_general(w0t_ref[...], out1t, (((0,), (1,)), ((), ())),
                        preferred_element_type=jnp.float32,
                        precision=lax.Precision.HIGHEST) * m
    c = lax.dot_general(b0_ref[...], out1t, (((0,), (1,)), ((), ())),
                        preferred_element_type=jnp.float32,
                        precision=lax.Precision.HIGHEST)[None, :] * m
    acc = jnp.dot(x_ref[0], n, preferred_element_type=jnp.float32)
    out = scale * (acc + c)
    out_ref[0] = out


def _logits_call(x, xg, maskf, W0, b0, W1, b1):
    return pl.pallas_call(
        _logits_body,
        grid=(B,),
        in_specs=[
            pl.BlockSpec((1, S, E), lambda b: (b, 0, 0)),
            pl.BlockSpec((1, NSLOT, E), lambda b: (b, 0, 0)),
            pl.BlockSpec((1, 1, NSLOT), lambda b: (b, 0, 0)),
            pl.BlockSpec((H, E), lambda b: (0, 0)),
            pl.BlockSpec((H,), lambda b: (0,)),
            pl.BlockSpec((H, E), lambda b: (0, 0)),
            pl.BlockSpec((H,), lambda b: (0,)),
        ],
        out_specs=pl.BlockSpec((1, S, NSLOT), lambda b: (b, 0, 0)),
        out_shape=jax.ShapeDtypeStruct((B, S, NSLOT), jnp.float32),
    )(x, xg, maskf, jnp.transpose(W0), b0, jnp.transpose(W1), b1)[:, :, :ROUT]


def kernel(x, sequences, W0, b0, W1, b1):
    flat_idx, maskf = _winner_call(sequences)
    xg = _gather_sc(x.reshape(B * S, E), flat_idx.reshape(B * NSLOT))
    return _logits_call(x, xg.reshape(B, NSLOT, E), maskf, W0, b0, W1, b1)
